# baseline (speedup 1.0000x reference)
import functools

import jax
import jax.numpy as jnp
from jax import lax
from jax.experimental import pallas as pl
from jax.experimental.pallas import tpu as pltpu
from jax.experimental.pallas import tpu_sc as plsc

D = 1024
S = 512
N_TOK = 32768
C = 32
NC = 2
NW = 32
TOK_PER_W = N_TOK // NW


def _ps_body(pos_ref, seg_ref, out_ref):
    out_ref[...] = pos_ref[...] + seg_ref[0]


def _build_ps(pos_table, seg_table):
    return pl.pallas_call(
        _ps_body,
        grid=(8,),
        in_specs=[
            pl.BlockSpec((128, D), lambda i: (i % 4, 0)),
            pl.BlockSpec((1, 1, D), lambda i: (i // 4, 0, 0)),
        ],
        out_specs=pl.BlockSpec((128, D), lambda i: (i, 0)),
        out_shape=jax.ShapeDtypeStruct((2 * S, D), jnp.float32),
    )(pos_table, seg_table.reshape(2, 1, D))


def _xlane_sum(x):
    for sh in (8, 4, 2, 1):
        idx = jnp.bitwise_xor(lax.iota(jnp.int32, 16), jnp.int32(sh))
        x = x + jnp.take_along_axis(x, idx, axis=0)
    return x


def _rsqrt_scalar(x):
    i = lax.bitcast_convert_type(x, jnp.int32)
    i = jnp.int32(0x5F3759DF) - lax.shift_right_logical(i, 1)
    y = lax.bitcast_convert_type(i, jnp.float32)
    for _ in range(3):
        y = y * (jnp.float32(1.5) - jnp.float32(0.5) * x * y * y)
    return y


def _sc_body(ids, tts, tok_tab, ps_tab, gam, bet, out,
             idx_v, cidx_v, tt_v, tokbuf, pbuf, gb_v, sem):
    wid = lax.axis_index("s") * NC + lax.axis_index("c")
    pltpu.sync_copy(gam, gb_v.at[0])
    pltpu.sync_copy(bet, gb_v.at[1])

    def chunk_body(c, _):
        base = wid * TOK_PER_W + c * C
        pltpu.sync_copy(ids.at[pl.ds(base, C)], idx_v)
        pltpu.sync_copy(tts.at[pl.ds(base, C)], tt_v)
        s_base = jnp.bitwise_and(c * C, S - 1)

        def mk_cidx(k, _):
            tt16 = tt_v[pl.ds(k * 16, 16)]
            s16 = lax.iota(jnp.int32, 16) + (s_base + k * 16)
            cidx_v[pl.ds(k * 16, 16)] = lax.shift_left(tt16, 9) + s16
            return 0
        lax.fori_loop(0, C // 16, mk_cidx, 0)

        cp1 = pltpu.async_copy(tok_tab.at[idx_v], tokbuf, sem)
        cp2 = pltpu.async_copy(ps_tab.at[cidx_v], pbuf, sem)
        cp1.wait()
        cp2.wait()

        def token_body(i, _):
            def p1(j, carry):
                s_acc, q_acc = carry
                t = tokbuf[i, pl.ds(j * 16, 16)]
                p = pbuf[i, pl.ds(j * 16, 16)]
                x = t + p
                tokbuf[i, pl.ds(j * 16, 16)] = x
                return (s_acc + x, q_acc + x * x)

            z = jnp.zeros((16,), jnp.float32)
            s_acc, q_acc = lax.fori_loop(0, D // 16, p1, (z, z))
            mean = _xlane_sum(s_acc)[0] * jnp.float32(1.0 / D)
            ex2 = _xlane_sum(q_acc)[0] * jnp.float32(1.0 / D)
            var = ex2 - mean * mean
            rstd = _rsqrt_scalar(var + jnp.float32(1e-12))
            a = jnp.full((16,), rstd, jnp.float32)
            b = jnp.full((16,), (jnp.float32(0.0) - mean) * rstd,
                         jnp.float32)

            def p2(j, _):
                x = tokbuf[i, pl.ds(j * 16, 16)]
                g = gb_v[0, pl.ds(j * 16, 16)]
                bt = gb_v[1, pl.ds(j * 16, 16)]
                tokbuf[i, pl.ds(j * 16, 16)] = (x * a + b) * g + bt
                return 0
            lax.fori_loop(0, D // 16, p2, 0)
            return 0
        lax.fori_loop(0, C, token_body, 0)

        pltpu.sync_copy(tokbuf, out.at[pl.ds(base, C)])
        return 0

    lax.fori_loop(0, TOK_PER_W // C, chunk_body, 0)


_sc_call = functools.partial(
    pl.kernel,
    mesh=plsc.VectorSubcoreMesh(core_axis_name="c", subcore_axis_name="s"),
    out_type=jax.ShapeDtypeStruct((N_TOK, D), jnp.float32),
    scratch_types=[
        pltpu.VMEM((C,), jnp.int32),
        pltpu.VMEM((C,), jnp.int32),
        pltpu.VMEM((C,), jnp.int32),
        pltpu.VMEM((C, D), jnp.float32),
        pltpu.VMEM((C, D), jnp.float32),
        pltpu.VMEM((2, D), jnp.float32),
        pltpu.SemaphoreType.DMA,
    ],
)(_sc_body)


def kernel(input_ids, token_type_ids, tok_table, pos_table, seg_table,
           ln_gamma, ln_beta):
    B, Sq = input_ids.shape
    ps = _build_ps(pos_table, seg_table)
    ids = input_ids.reshape(-1)
    tts = token_type_ids.reshape(-1)
    out = _sc_call(ids, tts, tok_table, ps, ln_gamma, ln_beta)
    return out.reshape(B, Sq, D)


# speedup vs baseline: 1.7015x; 1.7015x over previous
import functools

import jax
import jax.numpy as jnp
from jax import lax
from jax.experimental import pallas as pl
from jax.experimental.pallas import tpu as pltpu
from jax.experimental.pallas import tpu_sc as plsc

D = 1024
S = 512
N_TOK = 32768
C = 16
NC = 2
NW = 32
TOK_PER_W = N_TOK // NW
NCH = TOK_PER_W // C


def _ps_body(pos_ref, seg_ref, out_ref):
    out_ref[...] = pos_ref[...] + seg_ref[0]


def _build_ps(pos_table, seg_table):
    return pl.pallas_call(
        _ps_body,
        grid=(8,),
        in_specs=[
            pl.BlockSpec((128, D), lambda i: (i % 4, 0)),
            pl.BlockSpec((1, 1, D), lambda i: (i // 4, 0, 0)),
        ],
        out_specs=pl.BlockSpec((128, D), lambda i: (i, 0)),
        out_shape=jax.ShapeDtypeStruct((2 * S, D), jnp.float32),
    )(pos_table, seg_table.reshape(2, 1, D))


def _xlane_sum(x):
    for sh in (8, 4, 2, 1):
        idx = jnp.bitwise_xor(lax.iota(jnp.int32, 16), jnp.int32(sh))
        x = x + jnp.take_along_axis(x, idx, axis=0)
    return x


def _rsqrt_scalar(x):
    i = lax.bitcast_convert_type(x, jnp.int32)
    i = jnp.int32(0x5F3759DF) - lax.shift_right_logical(i, 1)
    y = lax.bitcast_convert_type(i, jnp.float32)
    for _ in range(3):
        y = y * (jnp.float32(1.5) - jnp.float32(0.5) * x * y * y)
    return y


def _sc_body(ids, tts, tok_tab, ps_tab, gam, bet, out,
             ids_v, tt_v, idx0, idx1, cidx0, cidx1,
             tok0, tok1, p0, p1, gb_v, sem0, sem1):
    wid = lax.axis_index("s") * NC + lax.axis_index("c")
    w_base = wid * TOK_PER_W
    pltpu.sync_copy(gam, gb_v.at[0])
    pltpu.sync_copy(bet, gb_v.at[1])
    pltpu.sync_copy(ids.at[pl.ds(w_base, TOK_PER_W)], ids_v)
    pltpu.sync_copy(tts.at[pl.ds(w_base, TOK_PER_W)], tt_v)

    idxs = (idx0, idx1)
    cidxs = (cidx0, cidx1)
    tokbufs = (tok0, tok1)
    pbufs = (p0, p1)
    sems = (sem0, sem1)
    lane = lax.iota(jnp.int32, 16)

    def issue(cc, b):
        off = cc * C
        idxs[b][...] = ids_v[pl.ds(off, C)]
        s16 = jnp.bitwise_and(off + lane, jnp.int32(S - 1))
        cidxs[b][...] = lax.shift_left(tt_v[pl.ds(off, C)], 9) + s16
        pltpu.async_copy(tok_tab.at[idxs[b]], tokbufs[b], sems[b])
        pltpu.async_copy(ps_tab.at[cidxs[b]], pbufs[b], sems[b])

    issue(0, 0)
    issue(1, 1)

    def outer(c2, _):
        for b in range(2):
            cc = c2 * 2 + b
            tokbuf = tokbufs[b]
            pbuf = pbufs[b]
            pltpu.make_async_copy(tok_tab.at[idxs[b]], tokbuf,
                                  sems[b]).wait()
            pltpu.make_async_copy(ps_tab.at[cidxs[b]], pbuf,
                                  sems[b]).wait()

            def token_body(i, _):
                s_acc = jnp.zeros((16,), jnp.float32)
                q_acc = jnp.zeros((16,), jnp.float32)
                for j in range(D // 16):
                    t = tokbuf[i, pl.ds(j * 16, 16)]
                    p = pbuf[i, pl.ds(j * 16, 16)]
                    x = t + p
                    tokbuf[i, pl.ds(j * 16, 16)] = x
                    s_acc = s_acc + x
                    q_acc = q_acc + x * x
                mean = _xlane_sum(s_acc)[0] * jnp.float32(1.0 / D)
                ex2 = _xlane_sum(q_acc)[0] * jnp.float32(1.0 / D)
                var = ex2 - mean * mean
                rstd = _rsqrt_scalar(var + jnp.float32(1e-12))
                a = jnp.full((16,), rstd, jnp.float32)
                bvec = jnp.full((16,), (jnp.float32(0.0) - mean) * rstd,
                                jnp.float32)
                for j in range(D // 16):
                    x = tokbuf[i, pl.ds(j * 16, 16)]
                    g = gb_v[0, pl.ds(j * 16, 16)]
                    bt = gb_v[1, pl.ds(j * 16, 16)]
                    tokbuf[i, pl.ds(j * 16, 16)] = (x * a + bvec) * g + bt
                return 0

            lax.fori_loop(0, C, token_body, 0)
            pltpu.sync_copy(tokbuf, out.at[pl.ds(w_base + cc * C, C)])

            @pl.when(cc + 2 < NCH)
            def _():
                issue(cc + 2, b)
        return 0

    lax.fori_loop(0, NCH // 2, outer, 0)


_sc_call = functools.partial(
    pl.kernel,
    mesh=plsc.VectorSubcoreMesh(core_axis_name="c", subcore_axis_name="s"),
    out_type=jax.ShapeDtypeStruct((N_TOK, D), jnp.float32),
    scratch_types=[
        pltpu.VMEM((TOK_PER_W,), jnp.int32),
        pltpu.VMEM((TOK_PER_W,), jnp.int32),
        pltpu.VMEM((C,), jnp.int32),
        pltpu.VMEM((C,), jnp.int32),
        pltpu.VMEM((C,), jnp.int32),
        pltpu.VMEM((C,), jnp.int32),
        pltpu.VMEM((C, D), jnp.float32),
        pltpu.VMEM((C, D), jnp.float32),
        pltpu.VMEM((C, D), jnp.float32),
        pltpu.VMEM((C, D), jnp.float32),
        pltpu.VMEM((2, D), jnp.float32),
        pltpu.SemaphoreType.DMA,
        pltpu.SemaphoreType.DMA,
    ],
)(_sc_body)


def kernel(input_ids, token_type_ids, tok_table, pos_table, seg_table,
           ln_gamma, ln_beta):
    B, Sq = input_ids.shape
    ps = _build_ps(pos_table, seg_table)
    ids = input_ids.reshape(-1)
    tts = token_type_ids.reshape(-1)
    out = _sc_call(ids, tts, tok_table, ps, ln_gamma, ln_beta)
    return out.reshape(B, Sq, D)


# speedup vs baseline: 2.0510x; 1.2054x over previous
import functools

import jax
import jax.numpy as jnp
from jax import lax
from jax.experimental import pallas as pl
from jax.experimental.pallas import tpu as pltpu
from jax.experimental.pallas import tpu_sc as plsc

D = 1024
S = 512
N_TOK = 32768
C = 16
NC = 2
NW = 32
TOK_PER_W = N_TOK // NW
NCH = TOK_PER_W // C
NBUF = 3


def _ps_body(pos_ref, seg_ref, out_ref):
    out_ref[...] = pos_ref[...] + seg_ref[0]


def _build_ps(pos_table, seg_table):
    return pl.pallas_call(
        _ps_body,
        grid=(8,),
        in_specs=[
            pl.BlockSpec((128, D), lambda i: (i % 4, 0)),
            pl.BlockSpec((1, 1, D), lambda i: (i // 4, 0, 0)),
        ],
        out_specs=pl.BlockSpec((128, D), lambda i: (i, 0)),
        out_shape=jax.ShapeDtypeStruct((2 * S, D), jnp.float32),
    )(pos_table, seg_table.reshape(2, 1, D))


def _xlane_sum(x):
    for sh in (8, 4, 2, 1):
        idx = jnp.bitwise_xor(lax.iota(jnp.int32, 16), jnp.int32(sh))
        x = x + jnp.take_along_axis(x, idx, axis=0)
    return x


def _rsqrt_scalar(x):
    i = lax.bitcast_convert_type(x, jnp.int32)
    i = jnp.int32(0x5F3759DF) - lax.shift_right_logical(i, 1)
    y = lax.bitcast_convert_type(i, jnp.float32)
    for _ in range(3):
        y = y * (jnp.float32(1.5) - jnp.float32(0.5) * x * y * y)
    return y


def _sc_body(ids, tts, tok_tab, ps_tab, gam, bet, out,
             ids_v, tt_v, idx3, cidx3, tok3, p3, gb_v, semg, sems):
    wid = lax.axis_index("s") * NC + lax.axis_index("c")
    w_base = wid * TOK_PER_W
    pltpu.sync_copy(gam, gb_v.at[0])
    pltpu.sync_copy(bet, gb_v.at[1])
    pltpu.sync_copy(ids.at[pl.ds(w_base, TOK_PER_W)], ids_v)
    pltpu.sync_copy(tts.at[pl.ds(w_base, TOK_PER_W)], tt_v)

    lane = lax.iota(jnp.int32, 16)

    def issue_g(cc, b):
        off = cc * C
        idx3[b, ...] = ids_v[pl.ds(off, C)]
        s16 = jnp.bitwise_and(off + lane, jnp.int32(S - 1))
        cidx3[b, ...] = lax.shift_left(tt_v[pl.ds(off, C)], 9) + s16
        pltpu.async_copy(tok_tab.at[idx3.at[b]], tok3.at[b], semg.at[b])
        pltpu.async_copy(ps_tab.at[cidx3.at[b]], p3.at[b], semg.at[b])

    def wait_g(b):
        pltpu.make_async_copy(tok_tab.at[idx3.at[b]], tok3.at[b],
                              semg.at[b]).wait()
        pltpu.make_async_copy(ps_tab.at[cidx3.at[b]], p3.at[b],
                              semg.at[b]).wait()

    def wait_s(b):
        pltpu.make_async_copy(tok3.at[b], out.at[pl.ds(w_base, C)],
                              sems.at[b]).wait()

    def compute_store(cc, b):
        @plsc.parallel_loop(0, C)
        def token_body(i):
            s_acc = jnp.zeros((16,), jnp.float32)
            q_acc = jnp.zeros((16,), jnp.float32)
            for j in range(D // 16):
                x = (tok3[b, i, pl.ds(j * 16, 16)]
                     + p3[b, i, pl.ds(j * 16, 16)])
                tok3[b, i, pl.ds(j * 16, 16)] = x
                s_acc = s_acc + x
                q_acc = q_acc + x * x
            mean = _xlane_sum(s_acc)[0] * jnp.float32(1.0 / D)
            ex2 = _xlane_sum(q_acc)[0] * jnp.float32(1.0 / D)
            var = ex2 - mean * mean
            rstd = _rsqrt_scalar(var + jnp.float32(1e-12))
            a = jnp.full((16,), rstd, jnp.float32)
            bvec = jnp.full((16,), (jnp.float32(0.0) - mean) * rstd,
                            jnp.float32)
            for j in range(D // 16):
                x = tok3[b, i, pl.ds(j * 16, 16)]
                g = gb_v[0, pl.ds(j * 16, 16)]
                bt = gb_v[1, pl.ds(j * 16, 16)]
                tok3[b, i, pl.ds(j * 16, 16)] = (x * a + bvec) * g + bt

        pltpu.async_copy(tok3.at[b], out.at[pl.ds(w_base + cc * C, C)],
                         sems.at[b])

    issue_g(0, 0)
    issue_g(1, 1)

    def chunk_body(cc, carry):
        b, b1, b2 = carry

        @pl.when(cc >= 1)
        def _():
            wait_s(b2)

        @pl.when(cc + 2 < NCH)
        def _():
            issue_g(cc + 2, b2)

        wait_g(b)
        compute_store(cc, b)
        return (b1, b2, b)

    lax.fori_loop(0, NCH, chunk_body,
                  (jnp.int32(0), jnp.int32(1), jnp.int32(2)))

    wait_s(jnp.int32((NCH - 1) % NBUF))


_sc_call = functools.partial(
    pl.kernel,
    mesh=plsc.VectorSubcoreMesh(core_axis_name="c", subcore_axis_name="s"),
    out_type=jax.ShapeDtypeStruct((N_TOK, D), jnp.float32),
    scratch_types=(
        [pltpu.VMEM((TOK_PER_W,), jnp.int32)] * 2
        + [pltpu.VMEM((NBUF, C), jnp.int32)] * 2
        + [pltpu.VMEM((NBUF, C, D), jnp.float32)] * 2
        + [pltpu.VMEM((2, D), jnp.float32)]
        + [pltpu.SemaphoreType.DMA((NBUF,))] * 2
    ),
)(_sc_body)


def kernel(input_ids, token_type_ids, tok_table, pos_table, seg_table,
           ln_gamma, ln_beta):
    B, Sq = input_ids.shape
    ps = _build_ps(pos_table, seg_table)
    ids = input_ids.reshape(-1)
    tts = token_type_ids.reshape(-1)
    out = _sc_call(ids, tts, tok_table, ps, ln_gamma, ln_beta)
    return out.reshape(B, Sq, D)


# speedup vs baseline: 2.9012x; 1.4145x over previous
import functools

import jax
import jax.numpy as jnp
from jax import lax
from jax.experimental import pallas as pl
from jax.experimental.pallas import tpu as pltpu
from jax.experimental.pallas import tpu_sc as plsc

D = 1024
S = 512
N_TOK = 32768
NC = 2
NW = 32
NSPLIT = 2
TOK_SPLIT = N_TOK // NSPLIT
TOK_PER_W = TOK_SPLIT // NW
C = 16
NCH = TOK_PER_W // C
NBUF = 3
R = 256


def _sc_gather_body(ids, tok_tab, out, ids_v, idx3, tok3, semg, sems):
    wid = lax.axis_index("s") * NC + lax.axis_index("c")
    w_base = wid * TOK_PER_W
    pltpu.sync_copy(ids.at[pl.ds(w_base, TOK_PER_W)], ids_v)

    def issue_g(cc, b):
        idx3[b, ...] = ids_v[pl.ds(cc * C, C)]
        pltpu.async_copy(tok_tab.at[idx3.at[b]], tok3.at[b], semg.at[b])

    def wait_g(b):
        pltpu.make_async_copy(tok_tab.at[idx3.at[b]], tok3.at[b],
                              semg.at[b]).wait()

    def wait_s(b):
        pltpu.make_async_copy(tok3.at[b], out.at[pl.ds(w_base, C)],
                              sems.at[b]).wait()

    issue_g(0, 0)
    issue_g(1, 1)

    def chunk_body(cc, carry):
        b, b1, b2 = carry

        @pl.when(cc + 2 < NCH)
        def _():
            issue_g(cc + 2, b2)

        wait_g(b)
        pltpu.sync_copy(tok3.at[b], out.at[pl.ds(w_base + cc * C, C)])
        return (b1, b2, b)

    lax.fori_loop(0, NCH, chunk_body,
                  (jnp.int32(0), jnp.int32(1), jnp.int32(2)))


_sc_gather = functools.partial(
    pl.kernel,
    mesh=plsc.VectorSubcoreMesh(core_axis_name="c", subcore_axis_name="s"),
    out_type=jax.ShapeDtypeStruct((TOK_SPLIT, D), jnp.float32),
    scratch_types=(
        [pltpu.VMEM((TOK_PER_W,), jnp.int32)]
        + [pltpu.VMEM((NBUF, C), jnp.int32)]
        + [pltpu.VMEM((NBUF, C, D), jnp.float32)]
        + [pltpu.SemaphoreType.DMA((NBUF,))] * 2
    ),
)(_sc_gather_body)


def _ln_body(emb_ref, pos_ref, seg_ref, ttf_ref, gam_ref, bet_ref, out_ref):
    s0 = seg_ref[0]
    sd = seg_ref[1] - seg_ref[0]
    x = emb_ref[...] + pos_ref[...] + s0 + ttf_ref[...] * sd
    mean = jnp.mean(x, axis=1, keepdims=True)
    ex2 = jnp.mean(x * x, axis=1, keepdims=True)
    var = ex2 - mean * mean
    rstd = lax.rsqrt(var + jnp.float32(1e-12))
    out_ref[...] = (x - mean) * rstd * gam_ref[...] + bet_ref[...]


def _tc_ln(emb, pos_table, seg_table, ttf, ln_gamma, ln_beta):
    grid = (TOK_SPLIT // R,)
    pos_blocks = S // R
    return pl.pallas_call(
        _ln_body,
        grid=grid,
        in_specs=[
            pl.BlockSpec((R, D), lambda i: (i, 0)),
            pl.BlockSpec((R, D), lambda i: (i % pos_blocks, 0)),
            pl.BlockSpec((2, 1, D), lambda i: (0, 0, 0)),
            pl.BlockSpec((R, 1), lambda i: (i, 0)),
            pl.BlockSpec((1, D), lambda i: (0, 0)),
            pl.BlockSpec((1, D), lambda i: (0, 0)),
        ],
        out_specs=pl.BlockSpec((R, D), lambda i: (i, 0)),
        out_shape=jax.ShapeDtypeStruct((TOK_SPLIT, D), jnp.float32),
    )(emb, pos_table, seg_table.reshape(2, 1, D), ttf,
      ln_gamma.reshape(1, D), ln_beta.reshape(1, D))


def kernel(input_ids, token_type_ids, tok_table, pos_table, seg_table,
           ln_gamma, ln_beta):
    B, Sq = input_ids.shape
    ids = input_ids.reshape(NSPLIT, TOK_SPLIT)
    ttf = token_type_ids.reshape(NSPLIT, TOK_SPLIT, 1).astype(jnp.float32)
    outs = []
    for h in range(NSPLIT):
        emb = _sc_gather(ids[h], tok_table)
        outs.append(_tc_ln(emb, pos_table, seg_table, ttf[h],
                           ln_gamma, ln_beta))
    return jnp.concatenate(outs, axis=0).reshape(B, Sq, D)


# speedup vs baseline: 3.4625x; 1.1935x over previous
import functools

import jax
import jax.numpy as jnp
from jax import lax
from jax.experimental import pallas as pl
from jax.experimental.pallas import tpu as pltpu
from jax.experimental.pallas import tpu_sc as plsc

D = 1024
S = 512
N_TOK = 32768
NC = 2
NW = 32
NSPLIT = 2
TOK_SPLIT = N_TOK // NSPLIT
TOK_PER_W = TOK_SPLIT // NW
C = 16
NCH = TOK_PER_W // C
NBUF = 3
R = 512


def _sc_gather_body(ids, tok_tab, out, ids_v, idx3, tok3, semg, sems):
    wid = lax.axis_index("s") * NC + lax.axis_index("c")
    w_base = wid * TOK_PER_W
    pltpu.sync_copy(ids.at[pl.ds(w_base, TOK_PER_W)], ids_v)

    def issue_g(cc, b):
        idx3[b, ...] = ids_v[pl.ds(cc * C, C)]
        pltpu.async_copy(tok_tab.at[idx3.at[b]], tok3.at[b], semg.at[b])

    def wait_g(b):
        pltpu.make_async_copy(tok_tab.at[idx3.at[b]], tok3.at[b],
                              semg.at[b]).wait()

    def wait_s(b):
        pltpu.make_async_copy(tok3.at[b], out.at[pl.ds(w_base, C)],
                              sems.at[b]).wait()

    issue_g(0, 0)
    issue_g(1, 1)

    def chunk_body(cc, carry):
        b, b1, b2 = carry

        @pl.when(cc + 2 < NCH)
        def _():
            issue_g(cc + 2, b2)

        wait_g(b)
        pltpu.sync_copy(tok3.at[b], out.at[pl.ds(w_base + cc * C, C)])
        return (b1, b2, b)

    lax.fori_loop(0, NCH, chunk_body,
                  (jnp.int32(0), jnp.int32(1), jnp.int32(2)))


_sc_gather = functools.partial(
    pl.kernel,
    mesh=plsc.VectorSubcoreMesh(core_axis_name="c", subcore_axis_name="s"),
    out_type=jax.ShapeDtypeStruct((TOK_SPLIT, D), jnp.float32),
    scratch_types=(
        [pltpu.VMEM((TOK_PER_W,), jnp.int32)]
        + [pltpu.VMEM((NBUF, C), jnp.int32)]
        + [pltpu.VMEM((NBUF, C, D), jnp.float32)]
        + [pltpu.SemaphoreType.DMA((NBUF,))] * 2
    ),
)(_sc_gather_body)


def _ln_body(emb_ref, pos_ref, seg_ref, ttf_ref, gam_ref, bet_ref, out_ref):
    s0 = seg_ref[0]
    sd = seg_ref[1] - seg_ref[0]
    x = emb_ref[...] + pos_ref[...] + s0 + ttf_ref[...] * sd
    mean = jnp.mean(x, axis=1, keepdims=True)
    ex2 = jnp.mean(x * x, axis=1, keepdims=True)
    var = ex2 - mean * mean
    rstd = lax.rsqrt(var + jnp.float32(1e-12))
    out_ref[...] = (x - mean) * rstd * gam_ref[...] + bet_ref[...]


def _tc_ln(emb, pos_table, seg_table, ttf, ln_gamma, ln_beta):
    grid = (TOK_SPLIT // R,)
    pos_blocks = S // R
    return pl.pallas_call(
        _ln_body,
        grid=grid,
        in_specs=[
            pl.BlockSpec((R, D), lambda i: (i, 0)),
            pl.BlockSpec((R, D), lambda i: (i % pos_blocks, 0)),
            pl.BlockSpec((2, 1, D), lambda i: (0, 0, 0)),
            pl.BlockSpec((R, 1), lambda i: (i, 0)),
            pl.BlockSpec((1, D), lambda i: (0, 0)),
            pl.BlockSpec((1, D), lambda i: (0, 0)),
        ],
        out_specs=pl.BlockSpec((R, D), lambda i: (i, 0)),
        out_shape=jax.ShapeDtypeStruct((TOK_SPLIT, D), jnp.float32),
    )(emb, pos_table, seg_table.reshape(2, 1, D), ttf,
      ln_gamma.reshape(1, D), ln_beta.reshape(1, D))


def kernel(input_ids, token_type_ids, tok_table, pos_table, seg_table,
           ln_gamma, ln_beta):
    B, Sq = input_ids.shape
    ids = input_ids.reshape(NSPLIT, TOK_SPLIT)
    ttf = token_type_ids.reshape(NSPLIT, TOK_SPLIT, 1).astype(jnp.float32)
    outs = []
    for h in range(NSPLIT):
        emb = _sc_gather(ids[h], tok_table)
        outs.append(_tc_ln(emb, pos_table, seg_table, ttf[h],
                           ln_gamma, ln_beta))
    return jnp.concatenate(outs, axis=0).reshape(B, Sq, D)
